# baseline (speedup 1.0000x reference)
import functools
import math

import jax
import jax.numpy as jnp
from jax import lax
from jax.experimental import pallas as pl
from jax.experimental.pallas import tpu as pltpu
from jax.experimental.pallas import tpu_sc as plsc

B, H, Q, S, D = 8, 16, 8, 4096, 128
K_KEEP = 2048
SINK = 4
K_CAND = K_KEEP - SINK


HPB = 4


def _attn_body(q_ref, k_ref, v_ref, o_ref, hs_ref, acc_ref):
    hg = pl.program_id(1)
    scale = 1.0 / math.sqrt(D)
    for u in range(HPB):
        q = q_ref[0, u]
        k = k_ref[0, u]
        v = v_ref[0, u]
        s = jnp.dot(q, k.T, preferred_element_type=jnp.float32) * scale
        m = jnp.max(s, axis=-1, keepdims=True)
        p = jnp.exp(s - m)
        l = jnp.sum(p, axis=-1, keepdims=True)
        w = p / l
        o_ref[0, u] = jnp.dot(w, v, preferred_element_type=jnp.float32)
        wsum = jnp.sum(w, axis=0, keepdims=True)
        acc_ref[pl.ds(hg * HPB + u, 1), :] = wsum

    @pl.when(hg == H // HPB - 1)
    def _():
        a = acc_ref[...]
        t = a[0:8] + a[8:16]
        t = t[0:4] + t[4:8]
        t = t[0:2] + t[2:4]
        hs_ref[0] = t[0:1] + t[1:2]


def _tc_attention(query, key, value):
    grid = (B, H // HPB)
    out = pl.pallas_call(
        _attn_body,
        grid=grid,
        in_specs=[
            pl.BlockSpec((1, HPB, Q, D), lambda b, h: (b, h, 0, 0)),
            pl.BlockSpec((1, HPB, S, D), lambda b, h: (b, h, 0, 0)),
            pl.BlockSpec((1, HPB, S, D), lambda b, h: (b, h, 0, 0)),
        ],
        out_specs=[
            pl.BlockSpec((1, HPB, Q, D), lambda b, h: (b, h, 0, 0)),
            pl.BlockSpec((1, 1, S), lambda b, h: (b, 0, 0)),
        ],
        out_shape=[
            jax.ShapeDtypeStruct((B, H, Q, D), jnp.float32),
            jax.ShapeDtypeStruct((B, 1, S), jnp.float32),
        ],
        scratch_shapes=[pltpu.VMEM((H, S), jnp.float32)],
    )(query, key, value)
    return out


def _select_body(hs_ref, cls_ref, aux_ref):
    hv = hs_ref[...]
    bits = lax.bitcast_convert_type(hv, jnp.int32)
    pos = (lax.broadcasted_iota(jnp.int32, hv.shape, 1) * 128
           + lax.broadcasted_iota(jnp.int32, hv.shape, 2))
    iscand = pos >= SINK

    def bs_body(_, lohi):
        lo, hi = lohi
        mid = lo + (hi - lo + 1) // 2
        cnt = jnp.sum(jnp.where(iscand & (bits >= mid), 1, 0),
                      axis=(1, 2), keepdims=True)
        take = cnt >= K_CAND
        return (jnp.where(take, mid, lo), jnp.where(take, hi, mid - 1))

    init = (jnp.zeros((B, 1, 1), jnp.int32),
            jnp.full((B, 1, 1), 2**31 - 2, jnp.int32))
    lo, _ = lax.fori_loop(0, 31, bs_body, init)
    t = lo
    gt = iscand & (bits > t)
    eq = iscand & (bits == t)
    g = jnp.sum(jnp.where(gt, 1, 0), axis=(1, 2), keepdims=True)
    r = K_CAND - g
    cls_ref[...] = jnp.where(pos < SINK, 2,
                             jnp.where(gt, 2, jnp.where(eq, 1, 0)))
    aux_ref[...] = jnp.broadcast_to(r, (B, 1, 128)).astype(jnp.int32)


def _tc_select(head_sum):
    SCH = S // 128
    out = pl.pallas_call(
        _select_body,
        out_shape=[
            jax.ShapeDtypeStruct((B, SCH, 128), jnp.int32),
            jax.ShapeDtypeStruct((B, 1, 128), jnp.int32),
        ],
    )(head_sum.reshape(B, SCH, 128))
    return out


NC, NS, L = 2, 16, 16
NW = NC * NS
PAIRS = (B * H) // NW
CHUNK = 128
NCHUNK = K_KEEP // CHUNK
NBUF = 3
LOOKA = 2


def _sc_evict(key_flat, value_flat, cls, aux):
    mesh = plsc.VectorSubcoreMesh(core_axis_name="c", subcore_axis_name="s")

    @functools.partial(
        pl.kernel,
        mesh=mesh,
        compiler_params=pltpu.CompilerParams(needs_layout_passes=False),
        out_type=[
            jax.ShapeDtypeStruct((B * H * K_KEEP, D), jnp.float32),
            jax.ShapeDtypeStruct((B * H * K_KEEP, D), jnp.float32),
        ],
        scratch_types=[
            pltpu.VMEM((S,), jnp.int32),
            pltpu.VMEM((128,), jnp.int32),
            pltpu.VMEM((K_KEEP + L,), jnp.int32),
            pltpu.VMEM((PAIRS * NCHUNK, CHUNK), jnp.int32),
        ] + [pltpu.VMEM((CHUNK, D), jnp.float32)] * (2 * NBUF)
          + [pltpu.SemaphoreType.DMA] * (4 * NBUF),
    )
    def body(key_hbm, value_hbm, cls_hbm, aux_hbm, outk_hbm, outv_hbm,
             cls_v, aux_v, idx_v, rows_v, *bufsem):
        bks = bufsem[0:NBUF]
        bvs = bufsem[NBUF:2 * NBUF]
        gsk = bufsem[2 * NBUF:3 * NBUF]
        gsv = bufsem[3 * NBUF:4 * NBUF]
        wsk = bufsem[4 * NBUF:5 * NBUF]
        wsv = bufsem[5 * NBUF:6 * NBUF]
        cid = lax.axis_index("c")
        sid = lax.axis_index("s")
        wid = sid * NC + cid
        b = wid % B
        hgrp = wid // B

        pltpu.sync_copy(cls_hbm.at[pl.ds(pl.multiple_of(b * S, S), S)], cls_v)
        pltpu.sync_copy(aux_hbm.at[pl.ds(pl.multiple_of(b * 128, 128), 128)],
                        aux_v)
        r = aux_v[pl.ds(0, L)][0]

        def comp_body(i, carry):
            nw_, nt_ = carry
            v = cls_v[pl.ds(i * L, L)]
            posv = i * L + lax.iota(jnp.int32, L)
            is2 = v == 2
            is1 = v == 1
            tp = plsc.cumsum(jnp.where(is1, 1, 0))
            take1 = is1 & ((nt_ + tp) <= r)
            keep = jnp.logical_or(is2, take1)
            plsc.store_compressed(idx_v.at[pl.ds(nw_, L)], posv, mask=keep)
            nk = plsc.all_reduce_population_count(keep)[0]
            ntk = plsc.all_reduce_population_count(take1)[0]
            return (nw_ + nk, nt_ + ntk)

        lax.fori_loop(0, S // L, comp_body, (jnp.int32(0), jnp.int32(0)))

        h0 = hgrp * PAIRS
        tbl_base = (b * H + h0) * S
        out_base = (b * H + h0) * K_KEEP
        npair_vregs = K_KEEP // L

        def rows_body(i, _):
            pair = i // npair_vregs
            m = i % npair_vregs
            c = pair * NCHUNK + m // (CHUNK // L)
            o = (m % (CHUNK // L)) * L
            rows_v[c, pl.ds(o, L)] = (idx_v[pl.ds(m * L, L)]
                                      + (tbl_base + pair * S))
            return 0

        lax.fori_loop(0, PAIRS * npair_vregs, rows_body, 0)

        NTOT = PAIRS * NCHUNK

        def _dst(c):
            pair = c // NCHUNK
            lc = c % NCHUNK
            off = out_base + pair * K_KEEP + lc * CHUNK
            return pl.ds(pl.multiple_of(off, CHUNK), CHUNK)

        def _wait(src, dstref, sem):
            pltpu.make_async_copy(src, dstref, sem).wait()

        def step(i, p):
            @pl.when(i < NTOT)
            def _():
                @pl.when(i >= NBUF)
                def _():
                    _wait(bks[p], outk_hbm.at[_dst(i - NBUF)], wsk[p])
                    _wait(bvs[p], outv_hbm.at[_dst(i - NBUF)], wsv[p])
                pltpu.async_copy(key_hbm.at[rows_v.at[i]], bks[p], gsk[p])
                pltpu.async_copy(value_hbm.at[rows_v.at[i]], bvs[p], gsv[p])

            j = i - LOOKA
            q = (p - LOOKA) % NBUF

            @pl.when(jnp.logical_and(j >= 0, j < NTOT))
            def _():
                _wait(key_hbm.at[pl.ds(0, CHUNK)], bks[q], gsk[q])
                _wait(value_hbm.at[pl.ds(0, CHUNK)], bvs[q], gsv[q])
                pltpu.async_copy(bks[q], outk_hbm.at[_dst(j)], wsk[q])
                pltpu.async_copy(bvs[q], outv_hbm.at[_dst(j)], wsv[q])

        def pipe_body(g, _):
            for p in range(NBUF):
                step(g * NBUF + p, p)
            return 0

        lax.fori_loop(0, -(-(NTOT + LOOKA) // NBUF), pipe_body, 0)
        for c in range(NTOT - NBUF, NTOT):
            _wait(bks[c % NBUF], outk_hbm.at[_dst(c)], wsk[c % NBUF])
            _wait(bvs[c % NBUF], outv_hbm.at[_dst(c)], wsv[c % NBUF])

    return body(key_flat, value_flat, cls, aux)


def kernel(query, key, value):
    attn_out, head_sum = _tc_attention(query, key, value)
    cls, aux = _tc_select(head_sum)
    key_flat = key.reshape(B * H * S, D)
    value_flat = value.reshape(B * H * S, D)
    outk, outv = _sc_evict(key_flat, value_flat,
                           cls.reshape(B * S), aux.reshape(B * 128))
    new_key = outk.reshape(B, H, K_KEEP, D)
    new_value = outv.reshape(B, H, K_KEEP, D)
    return attn_out, (new_key, new_value)
